# baseline (speedup 1.0000x reference)
import jax
import jax.numpy as jnp
from jax import lax
from jax.experimental import pallas as pl
from jax.experimental.pallas import tpu as pltpu
from jax.experimental.pallas import tpu_sc as plsc

N = 10000
D = 128
H = 128
C_OUT = 47
E = 320000
W2 = 64

NC = 2
NS = 16
NW = NC * NS
G = 128
RPW = 80
E_PAD = NW * RPW * G
N_PAD = 10240
RT = N_PAD // NS
MCH = 4
NM = RPW // MCH
NMP = NM // 2


def _fill(ref, nrows, ncols, value):
    v = jnp.full((16,), value, dtype=jnp.float32)
    nchunk = ncols // 16

    def body(i, carry):
        ref[i // nchunk, pl.ds((i % nchunk) * 16, 16)] = v
        return carry

    lax.fori_loop(0, nrows * nchunk, body, 0)


def _sc_agg(table, src2, dst2, with_deg):
    width = table.shape[1]

    def body(*refs):
        if with_deg:
            (src2_h, dst2_h, table_h, acc_o, deg_o,
             src_v, dst_v, rows_v, acc_sh,
             gsem0, gsem1, ssem0, ssem1, isem, ones_v, deg_sh, dsem) = refs
        else:
            (src2_h, dst2_h, table_h, acc_o,
             src_v, dst_v, rows_v, acc_sh,
             gsem0, gsem1, ssem0, ssem1, isem) = refs
        cid = lax.axis_index("c")
        sid = lax.axis_index("s")
        wid = cid * NS + sid
        gsem = (gsem0, gsem1)
        ssem = (ssem0, ssem1)

        def gather(idx_row, p):
            pltpu.async_copy(table_h.at[idx_row], rows_v.at[p], gsem[p])

        def wait_gather(p):
            pltpu.make_async_copy(table_h.at[src_v.at[0, 0]],
                                  rows_v.at[p], gsem[p]).wait()

        def scatter(p, idx_row):
            pltpu.async_copy(rows_v.at[p], acc_sh.at[idx_row], ssem[p],
                             add=True)

        def wait_scatter(p):
            pltpu.make_async_copy(rows_v.at[p], acc_sh.at[dst_v.at[0, 0]],
                                  ssem[p]).wait()

        _fill(rows_v.at[0], G, width, 0.0)
        for k in range(RT // G):
            pltpu.sync_copy(rows_v.at[0], acc_sh.at[pl.ds(sid * RT + k * G, G)])
        if with_deg:
            _fill(ones_v, G, 16, 0.0)
            for k in range(RT // G):
                pltpu.sync_copy(ones_v, deg_sh.at[pl.ds(sid * RT + k * G, G)])
            _fill(ones_v, G, 16, 1.0)

        plsc.subcore_barrier()

        def idx_off(m):
            return pl.multiple_of(wid * RPW + m * MCH, 4)

        pltpu.sync_copy(src2_h.at[pl.ds(idx_off(0), MCH)], src_v.at[0])
        pltpu.sync_copy(dst2_h.at[pl.ds(idx_off(0), MCH)], dst_v.at[0])
        gather(src_v.at[0, 0], 0)

        def half(t, hf):
            m = 2 * t + hf

            def drain_prev():
                wait_scatter(1)
                if with_deg:
                    for _ in range(MCH):
                        pltpu.make_async_copy(
                            ones_v, deg_sh.at[dst_v.at[1 - hf, 0]],
                            dsem).wait()

            if hf == 0:
                pl.when(t > 0)(drain_prev)
            else:
                drain_prev()

            def prefetch():
                pltpu.async_copy(src2_h.at[pl.ds(idx_off(m + 1), MCH)],
                                 src_v.at[1 - hf], isem)
                pltpu.async_copy(dst2_h.at[pl.ds(idx_off(m + 1), MCH)],
                                 dst_v.at[1 - hf], isem)

            if hf == 1:
                pl.when(t < NMP - 1)(prefetch)
            else:
                prefetch()

            for j in range(MCH):
                p = j % 2
                wait_gather(p)
                scatter(p, dst_v.at[hf, j])
                if with_deg:
                    pltpu.async_copy(ones_v, deg_sh.at[dst_v.at[hf, j]],
                                     dsem, add=True)
                if j > 0:
                    wait_scatter(1 - p)
                if j < MCH - 1:
                    gather(src_v.at[hf, j + 1], 1 - p)
                else:
                    def boundary():
                        pltpu.make_async_copy(
                            src2_h.at[pl.ds(idx_off(m + 1), MCH)],
                            src_v.at[1 - hf], isem).wait()
                        pltpu.make_async_copy(
                            dst2_h.at[pl.ds(idx_off(m + 1), MCH)],
                            dst_v.at[1 - hf], isem).wait()
                        gather(src_v.at[1 - hf, 0], 1 - p)
                    if hf == 1:
                        pl.when(t < NMP - 1)(boundary)
                    else:
                        boundary()

        def pair(t, carry):
            half(t, 0)
            half(t, 1)
            return carry

        lax.fori_loop(0, NMP, pair, 0)

        wait_scatter(1)
        if with_deg:
            for _ in range(MCH):
                pltpu.make_async_copy(ones_v, deg_sh.at[dst_v.at[1, 0]],
                                      dsem).wait()

        plsc.subcore_barrier()

        pltpu.sync_copy(acc_sh.at[pl.ds(sid * RT, RT)],
                        acc_o.at[cid, pl.ds(sid * RT, RT)])
        if with_deg:
            pltpu.sync_copy(deg_sh.at[pl.ds(sid * RT, RT)],
                            deg_o.at[cid, pl.ds(sid * RT, RT)])

    out_type = [jax.ShapeDtypeStruct((NC, N_PAD, width), jnp.float32)]
    scratch = [pltpu.VMEM((2, MCH, G), jnp.int32),
               pltpu.VMEM((2, MCH, G), jnp.int32),
               pltpu.VMEM((2, G, width), jnp.float32),
               pltpu.VMEM_SHARED((N_PAD, width), jnp.float32),
               pltpu.SemaphoreType.DMA,
               pltpu.SemaphoreType.DMA,
               pltpu.SemaphoreType.DMA,
               pltpu.SemaphoreType.DMA,
               pltpu.SemaphoreType.DMA]
    if with_deg:
        out_type.append(jax.ShapeDtypeStruct((NC, N_PAD, 16), jnp.float32))
        scratch += [pltpu.VMEM((G, 16), jnp.float32),
                    pltpu.VMEM_SHARED((N_PAD, 16), jnp.float32),
                    pltpu.SemaphoreType.DMA]

    fn = pl.kernel(
        body,
        out_type=out_type,
        mesh=plsc.VectorSubcoreMesh(core_axis_name="c", subcore_axis_name="s"),
        scratch_types=scratch,
        compiler_params=pltpu.CompilerParams(use_tc_tiling_on_sc=False),
    )
    return fn(src2, dst2, table)


R = 400
NB = N // R


def _tc_layer0(x, wn, ws, b):
    def body(x_r, wn_r, ws_r, b_r, xw_o, self_o):
        xv = x_r[...]
        xw_o[...] = jnp.dot(xv, wn_r[...], preferred_element_type=jnp.float32)
        self_o[...] = jnp.dot(xv, ws_r[...],
                              preferred_element_type=jnp.float32) + b_r[...]

    return pl.pallas_call(
        body,
        grid=(NB,),
        in_specs=[pl.BlockSpec((R, D), lambda i: (i, 0)),
                  pl.BlockSpec((D, H), lambda i: (0, 0)),
                  pl.BlockSpec((D, H), lambda i: (0, 0)),
                  pl.BlockSpec((1, H), lambda i: (0, 0))],
        out_specs=[pl.BlockSpec((R, H), lambda i: (i, 0)),
                   pl.BlockSpec((R, H), lambda i: (i, 0))],
        out_shape=[jax.ShapeDtypeStruct((N, H), jnp.float32),
                   jax.ShapeDtypeStruct((N, H), jnp.float32)],
    )(x, wn, ws, b.reshape(1, H))


def _tc_mid(self0, acc, deg, wn1p, ws1p, b1p):
    def body(s_r, a_r, d_r, wn_r, ws_r, b_r, hw_o, hs_o):
        degv = d_r[0, :, 0:1] + d_r[1, :, 0:1]
        accv = a_r[0] + a_r[1]
        h = s_r[...] + accv / jnp.maximum(degv, 1.0)
        h = jnp.maximum(h, 0.0)
        nrm = jnp.sqrt(jnp.sum(h * h, axis=1, keepdims=True))
        h = h / jnp.maximum(nrm, 1e-12)
        hw_o[...] = jnp.dot(h, wn_r[...], preferred_element_type=jnp.float32)
        hs_o[...] = jnp.dot(h, ws_r[...],
                            preferred_element_type=jnp.float32) + b_r[...]

    return pl.pallas_call(
        body,
        grid=(NB,),
        in_specs=[pl.BlockSpec((R, H), lambda i: (i, 0)),
                  pl.BlockSpec((NC, R, H), lambda i: (0, i, 0)),
                  pl.BlockSpec((NC, R, 16), lambda i: (0, i, 0)),
                  pl.BlockSpec((H, W2), lambda i: (0, 0)),
                  pl.BlockSpec((H, W2), lambda i: (0, 0)),
                  pl.BlockSpec((1, W2), lambda i: (0, 0))],
        out_specs=[pl.BlockSpec((R, W2), lambda i: (i, 0)),
                   pl.BlockSpec((R, W2), lambda i: (i, 0))],
        out_shape=[jax.ShapeDtypeStruct((N, W2), jnp.float32),
                   jax.ShapeDtypeStruct((N, W2), jnp.float32)],
    )(self0, acc, deg, wn1p, ws1p, b1p)


def _tc_final(hs1, acc, deg):
    def body(s_r, a_r, d_r, o_r):
        degv = d_r[0, :, 0:1] + d_r[1, :, 0:1]
        o_r[...] = s_r[...] + (a_r[0] + a_r[1]) / jnp.maximum(degv, 1.0)

    return pl.pallas_call(
        body,
        grid=(NB,),
        in_specs=[pl.BlockSpec((R, W2), lambda i: (i, 0)),
                  pl.BlockSpec((NC, R, W2), lambda i: (0, i, 0)),
                  pl.BlockSpec((NC, R, 16), lambda i: (0, i, 0))],
        out_specs=pl.BlockSpec((R, W2), lambda i: (i, 0)),
        out_shape=jax.ShapeDtypeStruct((N, W2), jnp.float32),
    )(hs1, acc, deg)


def kernel(features, edge_index, W_self0, W_neigh0, b0, W_self1, W_neigh1, b1):
    pad = E_PAD - E
    pad_src = jnp.zeros((pad,), jnp.int32)
    pad_dst = N + jnp.arange(pad, dtype=jnp.int32) % (N_PAD - N)
    src2 = jnp.concatenate([edge_index[0], pad_src]).reshape(E_PAD // G, G)
    dst2 = jnp.concatenate([edge_index[1], pad_dst]).reshape(E_PAD // G, G)
    wn1p = jnp.zeros((H, W2), jnp.float32).at[:, :C_OUT].set(W_neigh1)
    ws1p = jnp.zeros((H, W2), jnp.float32).at[:, :C_OUT].set(W_self1)
    b1p = jnp.zeros((1, W2), jnp.float32).at[0, :C_OUT].set(b1)

    xw0, self0 = _tc_layer0(features, W_neigh0, W_self0, b0)
    acc0, deg = _sc_agg(xw0, src2, dst2, with_deg=True)
    hw1, hs1 = _tc_mid(self0, acc0, deg, wn1p, ws1p, b1p)
    (acc1,) = _sc_agg(hw1, src2, dst2, with_deg=False)
    out64 = _tc_final(hs1, acc1, deg)
    return out64[:, :C_OUT]


# speedup vs baseline: 1.0333x; 1.0333x over previous
import jax
import jax.numpy as jnp
from jax import lax
from jax.experimental import pallas as pl
from jax.experimental.pallas import tpu as pltpu
from jax.experimental.pallas import tpu_sc as plsc

N = 10000
D = 128
H = 128
C_OUT = 47
E = 320000
W2 = 64

NC = 2
NS = 16
NW = NC * NS
G = 128
RPW = 80
E_PAD = NW * RPW * G
N_PAD = 10240
RT = N_PAD // NS
MCH = 4
NM = RPW // MCH
NMP = NM // 2


def _fill(ref, nrows, ncols, value):
    v = jnp.full((16,), value, dtype=jnp.float32)
    nchunk = ncols // 16

    def body(i, carry):
        ref[i // nchunk, pl.ds((i % nchunk) * 16, 16)] = v
        return carry

    lax.fori_loop(0, nrows * nchunk, body, 0)


def _sc_agg(table, src2, dst2, with_deg):
    width = table.shape[1]

    def body(*refs):
        if with_deg:
            (src2_h, dst2_h, table_h, acc_o, deg_o,
             src_v, dst_v, rows_v, acc_sh,
             gsem0, gsem1, ssem0, ssem1, isem, ones_v, deg_sh, dsem) = refs
        else:
            (src2_h, dst2_h, table_h, acc_o,
             src_v, dst_v, rows_v, acc_sh,
             gsem0, gsem1, ssem0, ssem1, isem) = refs
        cid = lax.axis_index("c")
        sid = lax.axis_index("s")
        wid = cid * NS + sid
        gsem = (gsem0, gsem1)
        ssem = (ssem0, ssem1)

        def gather(idx_row, p):
            pltpu.async_copy(table_h.at[idx_row], rows_v.at[p], gsem[p])

        def wait_gather(p):
            pltpu.make_async_copy(table_h.at[src_v.at[0, 0]],
                                  rows_v.at[p], gsem[p]).wait()

        def scatter(p, idx_row):
            pltpu.async_copy(rows_v.at[p], acc_sh.at[idx_row], ssem[p],
                             add=True)

        def wait_scatter(p):
            pltpu.make_async_copy(rows_v.at[p], acc_sh.at[dst_v.at[0, 0]],
                                  ssem[p]).wait()

        _fill(rows_v.at[0], G, width, 0.0)
        for k in range(RT // G):
            pltpu.sync_copy(rows_v.at[0], acc_sh.at[pl.ds(sid * RT + k * G, G)])
        if with_deg:
            _fill(ones_v, G, 16, 0.0)
            for k in range(RT // G):
                pltpu.sync_copy(ones_v, deg_sh.at[pl.ds(sid * RT + k * G, G)])
            _fill(ones_v, G, 16, 1.0)

        plsc.subcore_barrier()

        def idx_off(m):
            return pl.multiple_of(wid * RPW + m * MCH, 4)

        pltpu.sync_copy(src2_h.at[pl.ds(idx_off(0), MCH)], src_v.at[0])
        pltpu.sync_copy(dst2_h.at[pl.ds(idx_off(0), MCH)], dst_v.at[0])
        gather(src_v.at[0, 0], 0)

        def half(t, hf):
            m = 2 * t + hf

            def drain_prev():
                wait_scatter(1)
                if with_deg:
                    for _ in range(MCH):
                        pltpu.make_async_copy(
                            ones_v, deg_sh.at[dst_v.at[1 - hf, 0]],
                            dsem).wait()

            if hf == 0:
                pl.when(t > 0)(drain_prev)
            else:
                drain_prev()

            def prefetch():
                pltpu.async_copy(src2_h.at[pl.ds(idx_off(m + 1), MCH)],
                                 src_v.at[1 - hf], isem)
                pltpu.async_copy(dst2_h.at[pl.ds(idx_off(m + 1), MCH)],
                                 dst_v.at[1 - hf], isem)

            if hf == 1:
                pl.when(t < NMP - 1)(prefetch)
            else:
                prefetch()

            for j in range(MCH):
                p = j % 2
                wait_gather(p)
                scatter(p, dst_v.at[hf, j])
                if with_deg:
                    pltpu.async_copy(ones_v, deg_sh.at[dst_v.at[hf, j]],
                                     dsem, add=True)
                if j > 0:
                    wait_scatter(1 - p)
                if j < MCH - 1:
                    gather(src_v.at[hf, j + 1], 1 - p)
                else:
                    def boundary():
                        pltpu.make_async_copy(
                            src2_h.at[pl.ds(idx_off(m + 1), MCH)],
                            src_v.at[1 - hf], isem).wait()
                        pltpu.make_async_copy(
                            dst2_h.at[pl.ds(idx_off(m + 1), MCH)],
                            dst_v.at[1 - hf], isem).wait()
                        gather(src_v.at[1 - hf, 0], 1 - p)
                    if hf == 1:
                        pl.when(t < NMP - 1)(boundary)
                    else:
                        boundary()

        def pair(t, carry):
            half(t, 0)
            half(t, 1)
            return carry

        lax.fori_loop(0, NMP, pair, 0)

        wait_scatter(1)
        if with_deg:
            for _ in range(MCH):
                pltpu.make_async_copy(ones_v, deg_sh.at[dst_v.at[1, 0]],
                                      dsem).wait()

        plsc.subcore_barrier()

        pltpu.sync_copy(acc_sh.at[pl.ds(sid * RT, RT)],
                        acc_o.at[cid, pl.ds(sid * RT, RT)])
        if with_deg:
            pltpu.sync_copy(deg_sh.at[pl.ds(sid * RT, RT)],
                            deg_o.at[cid, pl.ds(sid * RT, RT)])

    out_type = [jax.ShapeDtypeStruct((NC, N_PAD, width), jnp.float32)]
    scratch = [pltpu.VMEM((2, MCH, G), jnp.int32),
               pltpu.VMEM((2, MCH, G), jnp.int32),
               pltpu.VMEM((2, G, width), jnp.float32),
               pltpu.VMEM_SHARED((N_PAD, width), jnp.float32),
               pltpu.SemaphoreType.DMA,
               pltpu.SemaphoreType.DMA,
               pltpu.SemaphoreType.DMA,
               pltpu.SemaphoreType.DMA,
               pltpu.SemaphoreType.DMA]
    if with_deg:
        out_type.append(jax.ShapeDtypeStruct((NC, N_PAD, 16), jnp.float32))
        scratch += [pltpu.VMEM((G, 16), jnp.float32),
                    pltpu.VMEM_SHARED((N_PAD, 16), jnp.float32),
                    pltpu.SemaphoreType.DMA]

    fn = pl.kernel(
        body,
        out_type=out_type,
        mesh=plsc.VectorSubcoreMesh(core_axis_name="c", subcore_axis_name="s"),
        scratch_types=scratch,
        compiler_params=pltpu.CompilerParams(use_tc_tiling_on_sc=False),
    )
    return fn(src2, dst2, table)


R = 400
NB = N // R


def _tc_layer0(x, wn, ws, b):
    def body(x_r, wn_r, ws_r, b_r, xw_o, self_o):
        xv = x_r[...]
        xw_o[...] = jnp.dot(xv, wn_r[...], preferred_element_type=jnp.float32)
        self_o[...] = jnp.dot(xv, ws_r[...],
                              preferred_element_type=jnp.float32) + b_r[...]

    return pl.pallas_call(
        body,
        grid=(NB,),
        in_specs=[pl.BlockSpec((R, D), lambda i: (i, 0)),
                  pl.BlockSpec((D, H), lambda i: (0, 0)),
                  pl.BlockSpec((D, H), lambda i: (0, 0)),
                  pl.BlockSpec((1, H), lambda i: (0, 0))],
        out_specs=[pl.BlockSpec((R, H), lambda i: (i, 0)),
                   pl.BlockSpec((R, H), lambda i: (i, 0))],
        out_shape=[jax.ShapeDtypeStruct((N, H), jnp.float32),
                   jax.ShapeDtypeStruct((N, H), jnp.float32)],
    )(x, wn, ws, b.reshape(1, H))


def _tc_mid(self0, acc, deg, wn1p, ws1p, b1p):
    def body(s_r, a_r, d_r, wn_r, ws_r, b_r, hw_o, hs_o):
        degv = d_r[0, :, 0:1] + d_r[1, :, 0:1]
        accv = a_r[0] + a_r[1]
        h = s_r[...] + accv / jnp.maximum(degv, 1.0)
        h = jnp.maximum(h, 0.0)
        nrm = jnp.sqrt(jnp.sum(h * h, axis=1, keepdims=True))
        h = h / jnp.maximum(nrm, 1e-12)
        hw_o[...] = jnp.dot(h, wn_r[...], preferred_element_type=jnp.float32)
        hs_o[...] = jnp.dot(h, ws_r[...],
                            preferred_element_type=jnp.float32) + b_r[...]

    return pl.pallas_call(
        body,
        grid=(NB,),
        in_specs=[pl.BlockSpec((R, H), lambda i: (i, 0)),
                  pl.BlockSpec((NC, R, H), lambda i: (0, i, 0)),
                  pl.BlockSpec((NC, R, 16), lambda i: (0, i, 0)),
                  pl.BlockSpec((H, W2), lambda i: (0, 0)),
                  pl.BlockSpec((H, W2), lambda i: (0, 0)),
                  pl.BlockSpec((1, W2), lambda i: (0, 0))],
        out_specs=[pl.BlockSpec((R, W2), lambda i: (i, 0)),
                   pl.BlockSpec((R, W2), lambda i: (i, 0))],
        out_shape=[jax.ShapeDtypeStruct((N, W2), jnp.float32),
                   jax.ShapeDtypeStruct((N, W2), jnp.float32)],
    )(self0, acc, deg, wn1p, ws1p, b1p)


def _tc_final(hs1, acc, deg):
    def body(s_r, a_r, d_r, o_r):
        degv = d_r[0, :, 0:1] + d_r[1, :, 0:1]
        o_r[...] = s_r[...] + (a_r[0] + a_r[1]) / jnp.maximum(degv, 1.0)

    return pl.pallas_call(
        body,
        grid=(NB,),
        in_specs=[pl.BlockSpec((R, W2), lambda i: (i, 0)),
                  pl.BlockSpec((NC, R, W2), lambda i: (0, i, 0)),
                  pl.BlockSpec((NC, R, 16), lambda i: (0, i, 0))],
        out_specs=pl.BlockSpec((R, W2), lambda i: (i, 0)),
        out_shape=jax.ShapeDtypeStruct((N, W2), jnp.float32),
    )(hs1, acc, deg)


def kernel(features, edge_index, W_self0, W_neigh0, b0, W_self1, W_neigh1, b1):
    ppw = (E_PAD - E) // NW
    pad_src = jnp.zeros((NW, ppw), jnp.int32)
    pad_dst = jnp.broadcast_to(N + jnp.arange(ppw, dtype=jnp.int32),
                               (NW, ppw))
    src2 = jnp.concatenate([edge_index[0].reshape(NW, E // NW), pad_src],
                           axis=1).reshape(E_PAD // G, G)
    dst2 = jnp.concatenate([edge_index[1].reshape(NW, E // NW), pad_dst],
                           axis=1).reshape(E_PAD // G, G)
    wn1p = jnp.zeros((H, W2), jnp.float32).at[:, :C_OUT].set(W_neigh1)
    ws1p = jnp.zeros((H, W2), jnp.float32).at[:, :C_OUT].set(W_self1)
    b1p = jnp.zeros((1, W2), jnp.float32).at[0, :C_OUT].set(b1)

    xw0, self0 = _tc_layer0(features, W_neigh0, W_self0, b0)
    acc0, deg = _sc_agg(xw0, src2, dst2, with_deg=True)
    hw1, hs1 = _tc_mid(self0, acc0, deg, wn1p, ws1p, b1p)
    (acc1,) = _sc_agg(hw1, src2, dst2, with_deg=False)
    out64 = _tc_final(hs1, acc1, deg)
    return out64[:, :C_OUT]


# speedup vs baseline: 1.0744x; 1.0397x over previous
import jax
import jax.numpy as jnp
from jax import lax
from jax.experimental import pallas as pl
from jax.experimental.pallas import tpu as pltpu
from jax.experimental.pallas import tpu_sc as plsc

N = 10000
D = 128
H = 128
C_OUT = 47
E = 320000
W2 = 64

NC = 2
NS = 16
NW = NC * NS
G = 128
RPW = 80
E_PAD = NW * RPW * G
N_PAD = 10240
RT = N_PAD // NS


def _fill(ref, nrows, ncols, value):
    v = jnp.full((16,), value, dtype=jnp.float32)
    nchunk = ncols // 16

    def body(i, carry):
        ref[i // nchunk, pl.ds((i % nchunk) * 16, 16)] = v
        return carry

    lax.fori_loop(0, nrows * nchunk, body, 0)


def _sc_agg(table, src2, dst2, with_deg, mch, nbuf):
    width = table.shape[1]
    nm = RPW // mch
    nmp = nm // 2

    def body(*refs):
        if with_deg:
            (src2_h, dst2_h, table_h, acc_o, deg_o,
             src_v, dst_v, rows_v, acc_sh) = refs[:9]
            gsem = refs[9:9 + nbuf]
            ssem = refs[9 + nbuf:9 + 2 * nbuf]
            isem = refs[9 + 2 * nbuf]
            ones_v, deg_sh, dsem = refs[10 + 2 * nbuf:]
        else:
            (src2_h, dst2_h, table_h, acc_o,
             src_v, dst_v, rows_v, acc_sh) = refs[:8]
            gsem = refs[8:8 + nbuf]
            ssem = refs[8 + nbuf:8 + 2 * nbuf]
            isem = refs[8 + 2 * nbuf]
        cid = lax.axis_index("c")
        sid = lax.axis_index("s")
        wid = cid * NS + sid

        def gather(idx_row, p):
            pltpu.async_copy(table_h.at[idx_row], rows_v.at[p], gsem[p])

        def wait_gather(p):
            pltpu.make_async_copy(table_h.at[src_v.at[0, 0]],
                                  rows_v.at[p], gsem[p]).wait()

        def scatter(p, idx_row):
            pltpu.async_copy(rows_v.at[p], acc_sh.at[idx_row], ssem[p],
                             add=True)

        def wait_scatter(p):
            pltpu.make_async_copy(rows_v.at[p], acc_sh.at[dst_v.at[0, 0]],
                                  ssem[p]).wait()

        def drain_deg():
            for _ in range(mch):
                pltpu.make_async_copy(ones_v, deg_sh.at[dst_v.at[0, 0]],
                                      dsem).wait()

        _fill(rows_v.at[0], G, width, 0.0)
        for k in range(RT // G):
            pltpu.sync_copy(rows_v.at[0], acc_sh.at[pl.ds(sid * RT + k * G, G)])
        if with_deg:
            _fill(ones_v, G, 16, 0.0)
            for k in range(RT // G):
                pltpu.sync_copy(ones_v, deg_sh.at[pl.ds(sid * RT + k * G, G)])
            _fill(ones_v, G, 16, 1.0)

        plsc.subcore_barrier()

        def idx_off(m):
            return pl.multiple_of(wid * RPW + m * mch, 4)

        pltpu.sync_copy(src2_h.at[pl.ds(idx_off(0), mch)], src_v.at[0])
        pltpu.sync_copy(dst2_h.at[pl.ds(idx_off(0), mch)], dst_v.at[0])
        for j in range(nbuf - 1):
            gather(src_v.at[0, j], j)

        def half(t, hf):
            m = 2 * t + hf

            def drain_prev():
                wait_scatter(nbuf - 1)
                if with_deg:
                    drain_deg()

            if hf == 0:
                pl.when(t > 0)(drain_prev)
            else:
                drain_prev()

            def prefetch():
                pltpu.async_copy(src2_h.at[pl.ds(idx_off(m + 1), mch)],
                                 src_v.at[1 - hf], isem)
                pltpu.async_copy(dst2_h.at[pl.ds(idx_off(m + 1), mch)],
                                 dst_v.at[1 - hf], isem)

            if hf == 1:
                pl.when(t < nmp - 1)(prefetch)
            else:
                prefetch()

            for j in range(mch):
                p = j % nbuf
                wait_gather(p)
                scatter(p, dst_v.at[hf, j])
                if with_deg:
                    pltpu.async_copy(ones_v, deg_sh.at[dst_v.at[hf, j]],
                                     dsem, add=True)
                q = (p + nbuf - 1) % nbuf
                if j > 0:
                    wait_scatter(q)
                jn = j + nbuf - 1
                if jn < mch:
                    gather(src_v.at[hf, jn], q)
                else:
                    if jn == mch:
                        def idx_arrived():
                            pltpu.make_async_copy(
                                src2_h.at[pl.ds(idx_off(m + 1), mch)],
                                src_v.at[1 - hf], isem).wait()
                            pltpu.make_async_copy(
                                dst2_h.at[pl.ds(idx_off(m + 1), mch)],
                                dst_v.at[1 - hf], isem).wait()
                        if hf == 1:
                            pl.when(t < nmp - 1)(idx_arrived)
                        else:
                            idx_arrived()

                    def boundary():
                        gather(src_v.at[1 - hf, jn - mch], q)
                    if hf == 1:
                        pl.when(t < nmp - 1)(boundary)
                    else:
                        boundary()

        def pair(t, carry):
            half(t, 0)
            half(t, 1)
            return carry

        lax.fori_loop(0, nmp, pair, 0)

        wait_scatter(nbuf - 1)
        if with_deg:
            drain_deg()

        plsc.subcore_barrier()

        pltpu.sync_copy(acc_sh.at[pl.ds(sid * RT, RT)],
                        acc_o.at[cid, pl.ds(sid * RT, RT)])
        if with_deg:
            pltpu.sync_copy(deg_sh.at[pl.ds(sid * RT, RT)],
                            deg_o.at[cid, pl.ds(sid * RT, RT)])

    out_type = [jax.ShapeDtypeStruct((NC, N_PAD, width), jnp.float32)]
    scratch = ([pltpu.VMEM((2, mch, G), jnp.int32),
                pltpu.VMEM((2, mch, G), jnp.int32),
                pltpu.VMEM((nbuf, G, width), jnp.float32),
                pltpu.VMEM_SHARED((N_PAD, width), jnp.float32)]
               + [pltpu.SemaphoreType.DMA] * (2 * nbuf + 1))
    if with_deg:
        out_type.append(jax.ShapeDtypeStruct((NC, N_PAD, 16), jnp.float32))
        scratch += [pltpu.VMEM((G, 16), jnp.float32),
                    pltpu.VMEM_SHARED((N_PAD, 16), jnp.float32),
                    pltpu.SemaphoreType.DMA]

    fn = pl.kernel(
        body,
        out_type=out_type,
        mesh=plsc.VectorSubcoreMesh(core_axis_name="c", subcore_axis_name="s"),
        scratch_types=scratch,
        compiler_params=pltpu.CompilerParams(use_tc_tiling_on_sc=False),
    )
    return fn(src2, dst2, table)


R = 400
NB = N // R


def _tc_layer0(x, wn, ws, b):
    def body(x_r, wn_r, ws_r, b_r, xw_o, self_o):
        xv = x_r[...]
        xw_o[...] = jnp.dot(xv, wn_r[...], preferred_element_type=jnp.float32)
        self_o[...] = jnp.dot(xv, ws_r[...],
                              preferred_element_type=jnp.float32) + b_r[...]

    return pl.pallas_call(
        body,
        grid=(NB,),
        in_specs=[pl.BlockSpec((R, D), lambda i: (i, 0)),
                  pl.BlockSpec((D, H), lambda i: (0, 0)),
                  pl.BlockSpec((D, H), lambda i: (0, 0)),
                  pl.BlockSpec((1, H), lambda i: (0, 0))],
        out_specs=[pl.BlockSpec((R, H), lambda i: (i, 0)),
                   pl.BlockSpec((R, H), lambda i: (i, 0))],
        out_shape=[jax.ShapeDtypeStruct((N, H), jnp.float32),
                   jax.ShapeDtypeStruct((N, H), jnp.float32)],
    )(x, wn, ws, b.reshape(1, H))


def _tc_mid(self0, acc, deg, wn1p, ws1p, b1p):
    def body(s_r, a_r, d_r, wn_r, ws_r, b_r, hw_o, hs_o):
        degv = d_r[0, :, 0:1] + d_r[1, :, 0:1]
        accv = a_r[0] + a_r[1]
        h = s_r[...] + accv / jnp.maximum(degv, 1.0)
        h = jnp.maximum(h, 0.0)
        nrm = jnp.sqrt(jnp.sum(h * h, axis=1, keepdims=True))
        h = h / jnp.maximum(nrm, 1e-12)
        hw_o[...] = jnp.dot(h, wn_r[...], preferred_element_type=jnp.float32)
        hs_o[...] = jnp.dot(h, ws_r[...],
                            preferred_element_type=jnp.float32) + b_r[...]

    return pl.pallas_call(
        body,
        grid=(NB,),
        in_specs=[pl.BlockSpec((R, H), lambda i: (i, 0)),
                  pl.BlockSpec((NC, R, H), lambda i: (0, i, 0)),
                  pl.BlockSpec((NC, R, 16), lambda i: (0, i, 0)),
                  pl.BlockSpec((H, W2), lambda i: (0, 0)),
                  pl.BlockSpec((H, W2), lambda i: (0, 0)),
                  pl.BlockSpec((1, W2), lambda i: (0, 0))],
        out_specs=[pl.BlockSpec((R, W2), lambda i: (i, 0)),
                   pl.BlockSpec((R, W2), lambda i: (i, 0))],
        out_shape=[jax.ShapeDtypeStruct((N, W2), jnp.float32),
                   jax.ShapeDtypeStruct((N, W2), jnp.float32)],
    )(self0, acc, deg, wn1p, ws1p, b1p)


def _tc_final(hs1, acc, deg):
    def body(s_r, a_r, d_r, o_r):
        degv = d_r[0, :, 0:1] + d_r[1, :, 0:1]
        o_r[...] = s_r[...] + (a_r[0] + a_r[1]) / jnp.maximum(degv, 1.0)

    return pl.pallas_call(
        body,
        grid=(NB,),
        in_specs=[pl.BlockSpec((R, W2), lambda i: (i, 0)),
                  pl.BlockSpec((NC, R, W2), lambda i: (0, i, 0)),
                  pl.BlockSpec((NC, R, 16), lambda i: (0, i, 0))],
        out_specs=pl.BlockSpec((R, W2), lambda i: (i, 0)),
        out_shape=jax.ShapeDtypeStruct((N, W2), jnp.float32),
    )(hs1, acc, deg)


def kernel(features, edge_index, W_self0, W_neigh0, b0, W_self1, W_neigh1, b1):
    ppw = (E_PAD - E) // NW
    pad_src = jnp.zeros((NW, ppw), jnp.int32)
    pad_dst = jnp.broadcast_to(N + jnp.arange(ppw, dtype=jnp.int32),
                               (NW, ppw))
    src2 = jnp.concatenate([edge_index[0].reshape(NW, E // NW), pad_src],
                           axis=1).reshape(E_PAD // G, G)
    dst2 = jnp.concatenate([edge_index[1].reshape(NW, E // NW), pad_dst],
                           axis=1).reshape(E_PAD // G, G)
    wn1p = jnp.zeros((H, W2), jnp.float32).at[:, :C_OUT].set(W_neigh1)
    ws1p = jnp.zeros((H, W2), jnp.float32).at[:, :C_OUT].set(W_self1)
    b1p = jnp.zeros((1, W2), jnp.float32).at[0, :C_OUT].set(b1)

    xw0, self0 = _tc_layer0(features, W_neigh0, W_self0, b0)
    acc0, deg = _sc_agg(xw0, src2, dst2, with_deg=True, mch=4, nbuf=2)
    hw1, hs1 = _tc_mid(self0, acc0, deg, wn1p, ws1p, b1p)
    (acc1,) = _sc_agg(hw1, src2, dst2, with_deg=False, mch=8, nbuf=4)
    out64 = _tc_final(hs1, acc1, deg)
    return out64[:, :C_OUT]


# speedup vs baseline: 1.1175x; 1.0401x over previous
import jax
import jax.numpy as jnp
from jax import lax
from jax.experimental import pallas as pl
from jax.experimental.pallas import tpu as pltpu
from jax.experimental.pallas import tpu_sc as plsc

N = 10000
D = 128
H = 128
C_OUT = 47
E = 320000
W2 = 64

NC = 2
NS = 16
NW = NC * NS
EPW = 10240
E_PAD = NW * EPW
N_PAD = 10240
RT = N_PAD // NS


def _fill(ref, nrows, ncols, value):
    v = jnp.full((16,), value, dtype=jnp.float32)
    nchunk = ncols // 16

    def body(i, carry):
        ref[i // nchunk, pl.ds((i % nchunk) * 16, 16)] = v
        return carry

    lax.fori_loop(0, nrows * nchunk, body, 0)


def _sc_agg(table, src2, dst2, with_deg, mch, nbuf, g):
    width = table.shape[1]
    rpw = EPW // g
    nm = rpw // mch
    nmp = nm // 2

    def body(*refs):
        if with_deg:
            (src2_h, dst2_h, table_h, acc_o, deg_o,
             src_v, dst_v, rows_v, acc_sh) = refs[:9]
            gsem = refs[9:9 + nbuf]
            ssem = refs[9 + nbuf:9 + 2 * nbuf]
            isem = refs[9 + 2 * nbuf]
            ones_v, deg_sh, dsem = refs[10 + 2 * nbuf:]
        else:
            (src2_h, dst2_h, table_h, acc_o,
             src_v, dst_v, rows_v, acc_sh) = refs[:8]
            gsem = refs[8:8 + nbuf]
            ssem = refs[8 + nbuf:8 + 2 * nbuf]
            isem = refs[8 + 2 * nbuf]
        cid = lax.axis_index("c")
        sid = lax.axis_index("s")
        wid = cid * NS + sid

        def gather(idx_row, p):
            pltpu.async_copy(table_h.at[idx_row], rows_v.at[p], gsem[p])

        def wait_gather(p):
            pltpu.make_async_copy(table_h.at[src_v.at[0, 0]],
                                  rows_v.at[p], gsem[p]).wait()

        def scatter(p, idx_row):
            pltpu.async_copy(rows_v.at[p], acc_sh.at[idx_row], ssem[p],
                             add=True)

        def wait_scatter(p):
            pltpu.make_async_copy(rows_v.at[p], acc_sh.at[dst_v.at[0, 0]],
                                  ssem[p]).wait()

        def drain_deg():
            for _ in range(mch):
                pltpu.make_async_copy(ones_v, deg_sh.at[dst_v.at[0, 0]],
                                      dsem).wait()

        _fill(rows_v.at[0], g, width, 0.0)
        for k in range(RT // g):
            pltpu.sync_copy(rows_v.at[0], acc_sh.at[pl.ds(sid * RT + k * g, g)])
        if with_deg:
            _fill(ones_v, g, 16, 0.0)
            for k in range(RT // g):
                pltpu.sync_copy(ones_v, deg_sh.at[pl.ds(sid * RT + k * g, g)])
            _fill(ones_v, g, 16, 1.0)

        plsc.subcore_barrier()

        def idx_off(m):
            return pl.multiple_of(wid * rpw + m * mch, 4)

        pltpu.sync_copy(src2_h.at[pl.ds(idx_off(0), mch)], src_v.at[0])
        pltpu.sync_copy(dst2_h.at[pl.ds(idx_off(0), mch)], dst_v.at[0])
        for j in range(nbuf - 1):
            gather(src_v.at[0, j], j)

        def half(t, hf):
            m = 2 * t + hf

            def drain_prev():
                wait_scatter(nbuf - 1)
                if with_deg:
                    drain_deg()

            if hf == 0:
                pl.when(t > 0)(drain_prev)
            else:
                drain_prev()

            def prefetch():
                pltpu.async_copy(src2_h.at[pl.ds(idx_off(m + 1), mch)],
                                 src_v.at[1 - hf], isem)
                pltpu.async_copy(dst2_h.at[pl.ds(idx_off(m + 1), mch)],
                                 dst_v.at[1 - hf], isem)

            if hf == 1:
                pl.when(t < nmp - 1)(prefetch)
            else:
                prefetch()

            for j in range(mch):
                p = j % nbuf
                wait_gather(p)
                scatter(p, dst_v.at[hf, j])
                if with_deg:
                    pltpu.async_copy(ones_v, deg_sh.at[dst_v.at[hf, j]],
                                     dsem, add=True)
                q = (p + nbuf - 1) % nbuf
                if j > 0:
                    wait_scatter(q)
                jn = j + nbuf - 1
                if jn < mch:
                    gather(src_v.at[hf, jn], q)
                else:
                    if jn == mch:
                        def idx_arrived():
                            pltpu.make_async_copy(
                                src2_h.at[pl.ds(idx_off(m + 1), mch)],
                                src_v.at[1 - hf], isem).wait()
                            pltpu.make_async_copy(
                                dst2_h.at[pl.ds(idx_off(m + 1), mch)],
                                dst_v.at[1 - hf], isem).wait()
                        if hf == 1:
                            pl.when(t < nmp - 1)(idx_arrived)
                        else:
                            idx_arrived()

                    def boundary():
                        gather(src_v.at[1 - hf, jn - mch], q)
                    if hf == 1:
                        pl.when(t < nmp - 1)(boundary)
                    else:
                        boundary()

        def pair(t, carry):
            half(t, 0)
            half(t, 1)
            return carry

        lax.fori_loop(0, nmp, pair, 0)

        wait_scatter(nbuf - 1)
        if with_deg:
            drain_deg()

        plsc.subcore_barrier()

        pltpu.sync_copy(acc_sh.at[pl.ds(sid * RT, RT)],
                        acc_o.at[cid, pl.ds(sid * RT, RT)])
        if with_deg:
            pltpu.sync_copy(deg_sh.at[pl.ds(sid * RT, RT)],
                            deg_o.at[cid, pl.ds(sid * RT, RT)])

    out_type = [jax.ShapeDtypeStruct((NC, N_PAD, width), jnp.float32)]
    scratch = ([pltpu.VMEM((2, mch, g), jnp.int32),
                pltpu.VMEM((2, mch, g), jnp.int32),
                pltpu.VMEM((nbuf, g, width), jnp.float32),
                pltpu.VMEM_SHARED((N_PAD, width), jnp.float32)]
               + [pltpu.SemaphoreType.DMA] * (2 * nbuf + 1))
    if with_deg:
        out_type.append(jax.ShapeDtypeStruct((NC, N_PAD, 16), jnp.float32))
        scratch += [pltpu.VMEM((g, 16), jnp.float32),
                    pltpu.VMEM_SHARED((N_PAD, 16), jnp.float32),
                    pltpu.SemaphoreType.DMA]

    fn = pl.kernel(
        body,
        out_type=out_type,
        mesh=plsc.VectorSubcoreMesh(core_axis_name="c", subcore_axis_name="s"),
        scratch_types=scratch,
        compiler_params=pltpu.CompilerParams(use_tc_tiling_on_sc=False),
    )
    return fn(src2, dst2, table)


R = 400
NB = N // R


def _tc_layer0(x, wn, ws, b):
    def body(x_r, wn_r, ws_r, b_r, xw_o, self_o):
        xv = x_r[...]
        xw_o[...] = jnp.dot(xv, wn_r[...], preferred_element_type=jnp.float32)
        self_o[...] = jnp.dot(xv, ws_r[...],
                              preferred_element_type=jnp.float32) + b_r[...]

    return pl.pallas_call(
        body,
        grid=(NB,),
        in_specs=[pl.BlockSpec((R, D), lambda i: (i, 0)),
                  pl.BlockSpec((D, H), lambda i: (0, 0)),
                  pl.BlockSpec((D, H), lambda i: (0, 0)),
                  pl.BlockSpec((1, H), lambda i: (0, 0))],
        out_specs=[pl.BlockSpec((R, H), lambda i: (i, 0)),
                   pl.BlockSpec((R, H), lambda i: (i, 0))],
        out_shape=[jax.ShapeDtypeStruct((N, H), jnp.float32),
                   jax.ShapeDtypeStruct((N, H), jnp.float32)],
    )(x, wn, ws, b.reshape(1, H))


def _tc_mid(self0, acc, deg, wn1p, ws1p, b1p):
    def body(s_r, a_r, d_r, wn_r, ws_r, b_r, hw_o, hs_o):
        degv = d_r[0, :, 0:1] + d_r[1, :, 0:1]
        accv = a_r[0] + a_r[1]
        h = s_r[...] + accv / jnp.maximum(degv, 1.0)
        h = jnp.maximum(h, 0.0)
        nrm = jnp.sqrt(jnp.sum(h * h, axis=1, keepdims=True))
        h = h / jnp.maximum(nrm, 1e-12)
        hw_o[...] = jnp.dot(h, wn_r[...], preferred_element_type=jnp.float32)
        hs_o[...] = jnp.dot(h, ws_r[...],
                            preferred_element_type=jnp.float32) + b_r[...]

    return pl.pallas_call(
        body,
        grid=(NB,),
        in_specs=[pl.BlockSpec((R, H), lambda i: (i, 0)),
                  pl.BlockSpec((NC, R, H), lambda i: (0, i, 0)),
                  pl.BlockSpec((NC, R, 16), lambda i: (0, i, 0)),
                  pl.BlockSpec((H, W2), lambda i: (0, 0)),
                  pl.BlockSpec((H, W2), lambda i: (0, 0)),
                  pl.BlockSpec((1, W2), lambda i: (0, 0))],
        out_specs=[pl.BlockSpec((R, W2), lambda i: (i, 0)),
                   pl.BlockSpec((R, W2), lambda i: (i, 0))],
        out_shape=[jax.ShapeDtypeStruct((N, W2), jnp.float32),
                   jax.ShapeDtypeStruct((N, W2), jnp.float32)],
    )(self0, acc, deg, wn1p, ws1p, b1p)


def _tc_final(hs1, acc, deg):
    def body(s_r, a_r, d_r, o_r):
        degv = d_r[0, :, 0:1] + d_r[1, :, 0:1]
        o_r[...] = s_r[...] + (a_r[0] + a_r[1]) / jnp.maximum(degv, 1.0)

    return pl.pallas_call(
        body,
        grid=(NB,),
        in_specs=[pl.BlockSpec((R, W2), lambda i: (i, 0)),
                  pl.BlockSpec((NC, R, W2), lambda i: (0, i, 0)),
                  pl.BlockSpec((NC, R, 16), lambda i: (0, i, 0))],
        out_specs=pl.BlockSpec((R, W2), lambda i: (i, 0)),
        out_shape=jax.ShapeDtypeStruct((N, W2), jnp.float32),
    )(hs1, acc, deg)


def kernel(features, edge_index, W_self0, W_neigh0, b0, W_self1, W_neigh1, b1):
    ppw = (E_PAD - E) // NW
    pad_src = jnp.zeros((NW, ppw), jnp.int32)
    pad_dst = jnp.broadcast_to(N + jnp.arange(ppw, dtype=jnp.int32),
                               (NW, ppw))
    src_p = jnp.concatenate([edge_index[0].reshape(NW, E // NW), pad_src],
                            axis=1)
    dst_p = jnp.concatenate([edge_index[1].reshape(NW, E // NW), pad_dst],
                            axis=1)
    wn1p = jnp.zeros((H, W2), jnp.float32).at[:, :C_OUT].set(W_neigh1)
    ws1p = jnp.zeros((H, W2), jnp.float32).at[:, :C_OUT].set(W_self1)
    b1p = jnp.zeros((1, W2), jnp.float32).at[0, :C_OUT].set(b1)

    xw0, self0 = _tc_layer0(features, W_neigh0, W_self0, b0)
    acc0, deg = _sc_agg(xw0, src_p.reshape(-1, 64), dst_p.reshape(-1, 64),
                        with_deg=True, mch=8, nbuf=4, g=64)
    hw1, hs1 = _tc_mid(self0, acc0, deg, wn1p, ws1p, b1p)
    (acc1,) = _sc_agg(hw1, src_p.reshape(-1, 128), dst_p.reshape(-1, 128),
                      with_deg=False, mch=8, nbuf=4, g=128)
    out64 = _tc_final(hs1, acc1, deg)
    return out64[:, :C_OUT]


# speedup vs baseline: 1.1178x; 1.0002x over previous
import jax
import jax.numpy as jnp
from jax import lax
from jax.experimental import pallas as pl
from jax.experimental.pallas import tpu as pltpu
from jax.experimental.pallas import tpu_sc as plsc

N = 10000
D = 128
H = 128
C_OUT = 47
E = 320000
W2 = 64

NC = 2
NS = 16
NW = NC * NS
EPW = 10240
E_PAD = NW * EPW
N_PAD = 10240
RT = N_PAD // NS


def _fill(ref, nrows, ncols, value):
    v = jnp.full((16,), value, dtype=jnp.float32)
    nchunk = ncols // 16

    def body(i, carry):
        ref[i // nchunk, pl.ds((i % nchunk) * 16, 16)] = v
        return carry

    lax.fori_loop(0, nrows * nchunk, body, 0)


def _sc_agg(table, src2, dst2, with_deg, mch, nbuf, g):
    width = table.shape[1]
    rpw = EPW // g
    nm = rpw // mch
    nmp = nm // 2

    def body(*refs):
        if with_deg:
            (src2_h, dst2_h, table_h, acc_o, deg_o,
             src_v, dst_v, rows_v, acc_sh) = refs[:9]
            gsem = refs[9:9 + nbuf]
            ssem = refs[9 + nbuf:9 + 2 * nbuf]
            isem = refs[9 + 2 * nbuf]
            ones_v, deg_sh, dsem = refs[10 + 2 * nbuf:]
        else:
            (src2_h, dst2_h, table_h, acc_o,
             src_v, dst_v, rows_v, acc_sh) = refs[:8]
            gsem = refs[8:8 + nbuf]
            ssem = refs[8 + nbuf:8 + 2 * nbuf]
            isem = refs[8 + 2 * nbuf]
        cid = lax.axis_index("c")
        sid = lax.axis_index("s")
        wid = cid * NS + sid

        def gather(idx_row, p):
            pltpu.async_copy(table_h.at[idx_row], rows_v.at[p], gsem[p])

        def wait_gather(p):
            pltpu.make_async_copy(table_h.at[src_v.at[0, 0]],
                                  rows_v.at[p], gsem[p]).wait()

        def scatter(p, idx_row):
            pltpu.async_copy(rows_v.at[p], acc_sh.at[idx_row], ssem[p],
                             add=True)

        def wait_scatter(p):
            pltpu.make_async_copy(rows_v.at[p], acc_sh.at[dst_v.at[0, 0]],
                                  ssem[p]).wait()

        def drain_deg():
            for _ in range(mch):
                pltpu.make_async_copy(ones_v, deg_sh.at[dst_v.at[0, 0]],
                                      dsem).wait()

        _fill(rows_v.at[0], g, width, 0.0)
        for k in range(RT // g):
            pltpu.sync_copy(rows_v.at[0], acc_sh.at[pl.ds(sid * RT + k * g, g)])
        if with_deg:
            _fill(ones_v, g, 16, 0.0)
            for k in range(RT // g):
                pltpu.sync_copy(ones_v, deg_sh.at[pl.ds(sid * RT + k * g, g)])
            _fill(ones_v, g, 16, 1.0)

        plsc.subcore_barrier()

        def idx_off(m):
            return pl.multiple_of(wid * rpw + m * mch, 4)

        pltpu.sync_copy(src2_h.at[pl.ds(idx_off(0), mch)], src_v.at[0])
        pltpu.sync_copy(dst2_h.at[pl.ds(idx_off(0), mch)], dst_v.at[0])
        for j in range(nbuf - 1):
            gather(src_v.at[0, j], j)

        def half(t, hf):
            m = 2 * t + hf

            def drain_prev():
                wait_scatter(nbuf - 1)
                if with_deg:
                    drain_deg()

            if hf == 0:
                pl.when(t > 0)(drain_prev)
            else:
                drain_prev()

            def prefetch():
                pltpu.async_copy(src2_h.at[pl.ds(idx_off(m + 1), mch)],
                                 src_v.at[1 - hf], isem)
                pltpu.async_copy(dst2_h.at[pl.ds(idx_off(m + 1), mch)],
                                 dst_v.at[1 - hf], isem)

            if hf == 1:
                pl.when(t < nmp - 1)(prefetch)
            else:
                prefetch()

            for j in range(mch):
                p = j % nbuf
                wait_gather(p)
                scatter(p, dst_v.at[hf, j])
                if with_deg:
                    pltpu.async_copy(ones_v, deg_sh.at[dst_v.at[hf, j]],
                                     dsem, add=True)
                q = (p + nbuf - 1) % nbuf
                if j > 0:
                    wait_scatter(q)
                jn = j + nbuf - 1
                if jn < mch:
                    gather(src_v.at[hf, jn], q)
                else:
                    if jn == mch:
                        def idx_arrived():
                            pltpu.make_async_copy(
                                src2_h.at[pl.ds(idx_off(m + 1), mch)],
                                src_v.at[1 - hf], isem).wait()
                            pltpu.make_async_copy(
                                dst2_h.at[pl.ds(idx_off(m + 1), mch)],
                                dst_v.at[1 - hf], isem).wait()
                        if hf == 1:
                            pl.when(t < nmp - 1)(idx_arrived)
                        else:
                            idx_arrived()

                    def boundary():
                        gather(src_v.at[1 - hf, jn - mch], q)
                    if hf == 1:
                        pl.when(t < nmp - 1)(boundary)
                    else:
                        boundary()

        def pair(t, carry):
            half(t, 0)
            half(t, 1)
            return carry

        lax.fori_loop(0, nmp, pair, 0)

        wait_scatter(nbuf - 1)
        if with_deg:
            drain_deg()

        plsc.subcore_barrier()

        pltpu.sync_copy(acc_sh.at[pl.ds(sid * RT, RT)],
                        acc_o.at[cid, pl.ds(sid * RT, RT)])
        if with_deg:
            pltpu.sync_copy(deg_sh.at[pl.ds(sid * RT, RT)],
                            deg_o.at[cid, pl.ds(sid * RT, RT)])

    out_type = [jax.ShapeDtypeStruct((NC, N_PAD, width), jnp.float32)]
    scratch = ([pltpu.VMEM((2, mch, g), jnp.int32),
                pltpu.VMEM((2, mch, g), jnp.int32),
                pltpu.VMEM((nbuf, g, width), jnp.float32),
                pltpu.VMEM_SHARED((N_PAD, width), jnp.float32)]
               + [pltpu.SemaphoreType.DMA] * (2 * nbuf + 1))
    if with_deg:
        out_type.append(jax.ShapeDtypeStruct((NC, N_PAD, 16), jnp.float32))
        scratch += [pltpu.VMEM((g, 16), jnp.float32),
                    pltpu.VMEM_SHARED((N_PAD, 16), jnp.float32),
                    pltpu.SemaphoreType.DMA]

    fn = pl.kernel(
        body,
        out_type=out_type,
        mesh=plsc.VectorSubcoreMesh(core_axis_name="c", subcore_axis_name="s"),
        scratch_types=scratch,
        compiler_params=pltpu.CompilerParams(use_tc_tiling_on_sc=False),
    )
    return fn(src2, dst2, table)


R = 400
NB = N // R


def _tc_layer0(x, wn, ws, b):
    def body(x_r, wn_r, ws_r, b_r, xw_o, self_o):
        xv = x_r[...]
        xw_o[...] = jnp.dot(xv, wn_r[...], preferred_element_type=jnp.float32)
        self_o[...] = jnp.dot(xv, ws_r[...],
                              preferred_element_type=jnp.float32) + b_r[...]

    return pl.pallas_call(
        body,
        grid=(NB,),
        in_specs=[pl.BlockSpec((R, D), lambda i: (i, 0)),
                  pl.BlockSpec((D, H), lambda i: (0, 0)),
                  pl.BlockSpec((D, H), lambda i: (0, 0)),
                  pl.BlockSpec((1, H), lambda i: (0, 0))],
        out_specs=[pl.BlockSpec((R, H), lambda i: (i, 0)),
                   pl.BlockSpec((R, H), lambda i: (i, 0))],
        out_shape=[jax.ShapeDtypeStruct((N, H), jnp.float32),
                   jax.ShapeDtypeStruct((N, H), jnp.float32)],
    )(x, wn, ws, b.reshape(1, H))


def _tc_mid(self0, acc, deg, wn1p, ws1p, b1p):
    def body(s_r, a_r, d_r, wn_r, ws_r, b_r, hw_o, hs_o):
        degv = d_r[0, :, 0:1] + d_r[1, :, 0:1]
        accv = a_r[0] + a_r[1]
        h = s_r[...] + accv / jnp.maximum(degv, 1.0)
        h = jnp.maximum(h, 0.0)
        nrm = jnp.sqrt(jnp.sum(h * h, axis=1, keepdims=True))
        h = h / jnp.maximum(nrm, 1e-12)
        hw_o[...] = jnp.dot(h, wn_r[...], preferred_element_type=jnp.float32)
        hs_o[...] = jnp.dot(h, ws_r[...],
                            preferred_element_type=jnp.float32) + b_r[...]

    return pl.pallas_call(
        body,
        grid=(NB,),
        in_specs=[pl.BlockSpec((R, H), lambda i: (i, 0)),
                  pl.BlockSpec((NC, R, H), lambda i: (0, i, 0)),
                  pl.BlockSpec((NC, R, 16), lambda i: (0, i, 0)),
                  pl.BlockSpec((H, W2), lambda i: (0, 0)),
                  pl.BlockSpec((H, W2), lambda i: (0, 0)),
                  pl.BlockSpec((1, W2), lambda i: (0, 0))],
        out_specs=[pl.BlockSpec((R, W2), lambda i: (i, 0)),
                   pl.BlockSpec((R, W2), lambda i: (i, 0))],
        out_shape=[jax.ShapeDtypeStruct((N, W2), jnp.float32),
                   jax.ShapeDtypeStruct((N, W2), jnp.float32)],
    )(self0, acc, deg, wn1p, ws1p, b1p)


def _tc_final(hs1, acc, deg):
    def body(s_r, a_r, d_r, o_r):
        degv = d_r[0, :, 0:1] + d_r[1, :, 0:1]
        o_r[...] = s_r[...] + (a_r[0] + a_r[1]) / jnp.maximum(degv, 1.0)

    return pl.pallas_call(
        body,
        grid=(NB,),
        in_specs=[pl.BlockSpec((R, W2), lambda i: (i, 0)),
                  pl.BlockSpec((NC, R, W2), lambda i: (0, i, 0)),
                  pl.BlockSpec((NC, R, 16), lambda i: (0, i, 0))],
        out_specs=pl.BlockSpec((R, W2), lambda i: (i, 0)),
        out_shape=jax.ShapeDtypeStruct((N, W2), jnp.float32),
    )(hs1, acc, deg)


def kernel(features, edge_index, W_self0, W_neigh0, b0, W_self1, W_neigh1, b1):
    ppw = (E_PAD - E) // NW
    pad_src = jnp.zeros((NW, ppw), jnp.int32)
    pad_dst = jnp.broadcast_to(N + jnp.arange(ppw, dtype=jnp.int32),
                               (NW, ppw))
    src_p = jnp.concatenate([edge_index[0].reshape(NW, E // NW), pad_src],
                            axis=1)
    dst_p = jnp.concatenate([edge_index[1].reshape(NW, E // NW), pad_dst],
                            axis=1)
    wn1p = jnp.zeros((H, W2), jnp.float32).at[:, :C_OUT].set(W_neigh1)
    ws1p = jnp.zeros((H, W2), jnp.float32).at[:, :C_OUT].set(W_self1)
    b1p = jnp.zeros((1, W2), jnp.float32).at[0, :C_OUT].set(b1)

    xw0, self0 = _tc_layer0(features, W_neigh0, W_self0, b0)
    acc0, deg = _sc_agg(xw0, src_p.reshape(-1, 64), dst_p.reshape(-1, 64),
                        with_deg=True, mch=8, nbuf=4, g=64)
    hw1, hs1 = _tc_mid(self0, acc0, deg, wn1p, ws1p, b1p)
    (acc1,) = _sc_agg(hw1, src_p.reshape(-1, 64), dst_p.reshape(-1, 64),
                      with_deg=False, mch=8, nbuf=8, g=64)
    out64 = _tc_final(hs1, acc1, deg)
    return out64[:, :C_OUT]


# speedup vs baseline: 1.1467x; 1.0258x over previous
import jax
import jax.numpy as jnp
from jax import lax
from jax.experimental import pallas as pl
from jax.experimental.pallas import tpu as pltpu
from jax.experimental.pallas import tpu_sc as plsc

N = 10000
D = 128
H = 128
C_OUT = 47
E = 320000
W2 = 64

NC = 2
NS = 16
NW = NC * NS
EPW = 10240
E_PAD = NW * EPW
N_PAD = 10240
RT = N_PAD // NS


def _fill(ref, nrows, ncols, value):
    v = jnp.full((16,), value, dtype=jnp.float32)
    nchunk = ncols // 16

    def body(i, carry):
        ref[i // nchunk, pl.ds((i % nchunk) * 16, 16)] = v
        return carry

    lax.fori_loop(0, nrows * nchunk, body, 0)


def _sc_agg(table, src2, dst2, with_deg, mch, nbuf, g):
    width = table.shape[1]
    rpw = EPW // g
    nm = rpw // mch
    nmp = nm // 2

    def body(*refs):
        if with_deg:
            (src2_h, dst2_h, table_h, acc_o, deg_o,
             src_v, dst_v, rows_v, acc_sh) = refs[:9]
            gsem = refs[9:9 + nbuf]
            ssem = refs[9 + nbuf:9 + 2 * nbuf]
            isem = refs[9 + 2 * nbuf]
            ones_v, deg_sh, dsem = refs[10 + 2 * nbuf:]
        else:
            (src2_h, dst2_h, table_h, acc_o,
             src_v, dst_v, rows_v, acc_sh) = refs[:8]
            gsem = refs[8:8 + nbuf]
            ssem = refs[8 + nbuf:8 + 2 * nbuf]
            isem = refs[8 + 2 * nbuf]
        cid = lax.axis_index("c")
        sid = lax.axis_index("s")
        wid = cid * NS + sid

        def gather(idx_row, p):
            pltpu.async_copy(table_h.at[idx_row], rows_v.at[p], gsem[p])

        def wait_gather(p):
            pltpu.make_async_copy(table_h.at[src_v.at[0, 0]],
                                  rows_v.at[p], gsem[p]).wait()

        def scatter(p, idx_row):
            pltpu.async_copy(rows_v.at[p], acc_sh.at[idx_row], ssem[p],
                             add=True)

        def wait_scatter(p):
            pltpu.make_async_copy(rows_v.at[p], acc_sh.at[dst_v.at[0, 0]],
                                  ssem[p]).wait()

        def drain_deg():
            for _ in range(mch):
                pltpu.make_async_copy(ones_v, deg_sh.at[dst_v.at[0, 0]],
                                      dsem).wait()

        _fill(rows_v.at[0], g, width, 0.0)
        for k in range(RT // g):
            pltpu.sync_copy(rows_v.at[0], acc_sh.at[pl.ds(sid * RT + k * g, g)])
        if with_deg:
            _fill(ones_v, g, 16, 0.0)
            for k in range(RT // g):
                pltpu.sync_copy(ones_v, deg_sh.at[pl.ds(sid * RT + k * g, g)])
            _fill(ones_v, g, 16, 1.0)

        plsc.subcore_barrier()

        def idx_off(m):
            return pl.multiple_of(wid * rpw + m * mch, 4)

        pltpu.sync_copy(src2_h.at[pl.ds(idx_off(0), mch)], src_v.at[0])
        pltpu.sync_copy(dst2_h.at[pl.ds(idx_off(0), mch)], dst_v.at[0])
        for j in range(nbuf - 1):
            gather(src_v.at[0, j], j)

        def half(t, hf):
            m = 2 * t + hf

            def drain_prev():
                wait_scatter(nbuf - 1)
                if with_deg:
                    drain_deg()

            if hf == 0:
                pl.when(t > 0)(drain_prev)
            else:
                drain_prev()

            def prefetch():
                pltpu.async_copy(src2_h.at[pl.ds(idx_off(m + 1), mch)],
                                 src_v.at[1 - hf], isem)
                pltpu.async_copy(dst2_h.at[pl.ds(idx_off(m + 1), mch)],
                                 dst_v.at[1 - hf], isem)

            if hf == 1:
                pl.when(t < nmp - 1)(prefetch)
            else:
                prefetch()

            for j in range(mch):
                p = j % nbuf
                wait_gather(p)
                scatter(p, dst_v.at[hf, j])
                if with_deg:
                    pltpu.async_copy(ones_v, deg_sh.at[dst_v.at[hf, j]],
                                     dsem, add=True)
                q = (p + nbuf - 1) % nbuf
                if j > 0:
                    wait_scatter(q)
                jn = j + nbuf - 1
                if jn < mch:
                    gather(src_v.at[hf, jn], q)
                else:
                    if jn == mch:
                        def idx_arrived():
                            pltpu.make_async_copy(
                                src2_h.at[pl.ds(idx_off(m + 1), mch)],
                                src_v.at[1 - hf], isem).wait()
                            pltpu.make_async_copy(
                                dst2_h.at[pl.ds(idx_off(m + 1), mch)],
                                dst_v.at[1 - hf], isem).wait()
                        if hf == 1:
                            pl.when(t < nmp - 1)(idx_arrived)
                        else:
                            idx_arrived()

                    def boundary():
                        gather(src_v.at[1 - hf, jn - mch], q)
                    if hf == 1:
                        pl.when(t < nmp - 1)(boundary)
                    else:
                        boundary()

        def pair(t, carry):
            half(t, 0)
            half(t, 1)
            return carry

        lax.fori_loop(0, nmp, pair, 0)

        wait_scatter(nbuf - 1)
        if with_deg:
            drain_deg()

        plsc.subcore_barrier()

        pltpu.sync_copy(acc_sh.at[pl.ds(sid * RT, RT)],
                        acc_o.at[cid, pl.ds(sid * RT, RT)])
        if with_deg:
            pltpu.sync_copy(deg_sh.at[pl.ds(sid * RT, RT)],
                            deg_o.at[cid, pl.ds(sid * RT, RT)])

    out_type = [jax.ShapeDtypeStruct((NC, N_PAD, width), jnp.float32)]
    scratch = ([pltpu.VMEM((2, mch, g), jnp.int32),
                pltpu.VMEM((2, mch, g), jnp.int32),
                pltpu.VMEM((nbuf, g, width), jnp.float32),
                pltpu.VMEM_SHARED((N_PAD, width), jnp.float32)]
               + [pltpu.SemaphoreType.DMA] * (2 * nbuf + 1))
    if with_deg:
        out_type.append(jax.ShapeDtypeStruct((NC, N_PAD, 16), jnp.float32))
        scratch += [pltpu.VMEM((g, 16), jnp.float32),
                    pltpu.VMEM_SHARED((N_PAD, 16), jnp.float32),
                    pltpu.SemaphoreType.DMA]

    fn = pl.kernel(
        body,
        out_type=out_type,
        mesh=plsc.VectorSubcoreMesh(core_axis_name="c", subcore_axis_name="s"),
        scratch_types=scratch,
        compiler_params=pltpu.CompilerParams(use_tc_tiling_on_sc=False),
    )
    return fn(src2, dst2, table)


R = 400
NB = N // R


def _tc_mid(x, acc, deg, ws0, wn0, b0, wn1p, ws1p, b1p):
    def body(x_r, a_r, d_r, ws0_r, wn0_r, b0_r, wn_r, ws_r, b_r, hw_o, hs_o):
        degv = d_r[0, :, 0:1] + d_r[1, :, 0:1]
        mean = (a_r[0] + a_r[1]) / jnp.maximum(degv, 1.0)
        h = (jnp.dot(x_r[...], ws0_r[...], preferred_element_type=jnp.float32)
             + jnp.dot(mean, wn0_r[...], preferred_element_type=jnp.float32)
             + b0_r[...])
        h = jnp.maximum(h, 0.0)
        nrm = jnp.sqrt(jnp.sum(h * h, axis=1, keepdims=True))
        h = h / jnp.maximum(nrm, 1e-12)
        hw_o[...] = jnp.dot(h, wn_r[...], preferred_element_type=jnp.float32)
        hs_o[...] = jnp.dot(h, ws_r[...],
                            preferred_element_type=jnp.float32) + b_r[...]

    return pl.pallas_call(
        body,
        grid=(NB,),
        in_specs=[pl.BlockSpec((R, D), lambda i: (i, 0)),
                  pl.BlockSpec((NC, R, D), lambda i: (0, i, 0)),
                  pl.BlockSpec((NC, R, 16), lambda i: (0, i, 0)),
                  pl.BlockSpec((D, H), lambda i: (0, 0)),
                  pl.BlockSpec((D, H), lambda i: (0, 0)),
                  pl.BlockSpec((1, H), lambda i: (0, 0)),
                  pl.BlockSpec((H, W2), lambda i: (0, 0)),
                  pl.BlockSpec((H, W2), lambda i: (0, 0)),
                  pl.BlockSpec((1, W2), lambda i: (0, 0))],
        out_specs=[pl.BlockSpec((R, W2), lambda i: (i, 0)),
                   pl.BlockSpec((R, W2), lambda i: (i, 0))],
        out_shape=[jax.ShapeDtypeStruct((N, W2), jnp.float32),
                   jax.ShapeDtypeStruct((N, W2), jnp.float32)],
    )(x, acc, deg, ws0, wn0, b0, wn1p, ws1p, b1p)


def _tc_final(hs1, acc, deg):
    def body(s_r, a_r, d_r, o_r):
        degv = d_r[0, :, 0:1] + d_r[1, :, 0:1]
        o_r[...] = s_r[...] + (a_r[0] + a_r[1]) / jnp.maximum(degv, 1.0)

    return pl.pallas_call(
        body,
        grid=(NB,),
        in_specs=[pl.BlockSpec((R, W2), lambda i: (i, 0)),
                  pl.BlockSpec((NC, R, W2), lambda i: (0, i, 0)),
                  pl.BlockSpec((NC, R, 16), lambda i: (0, i, 0))],
        out_specs=pl.BlockSpec((R, W2), lambda i: (i, 0)),
        out_shape=jax.ShapeDtypeStruct((N, W2), jnp.float32),
    )(hs1, acc, deg)


def kernel(features, edge_index, W_self0, W_neigh0, b0, W_self1, W_neigh1, b1):
    ppw = (E_PAD - E) // NW
    pad_src = jnp.zeros((NW, ppw), jnp.int32)
    pad_dst = jnp.broadcast_to(N + jnp.arange(ppw, dtype=jnp.int32),
                               (NW, ppw))
    src_p = jnp.concatenate([edge_index[0].reshape(NW, E // NW), pad_src],
                            axis=1)
    dst_p = jnp.concatenate([edge_index[1].reshape(NW, E // NW), pad_dst],
                            axis=1)
    wn1p = jnp.zeros((H, W2), jnp.float32).at[:, :C_OUT].set(W_neigh1)
    ws1p = jnp.zeros((H, W2), jnp.float32).at[:, :C_OUT].set(W_self1)
    b1p = jnp.zeros((1, W2), jnp.float32).at[0, :C_OUT].set(b1)

    acc0, deg = _sc_agg(features, src_p.reshape(-1, 64), dst_p.reshape(-1, 64),
                        with_deg=True, mch=8, nbuf=4, g=64)
    hw1, hs1 = _tc_mid(features, acc0, deg, W_self0, W_neigh0,
                       b0.reshape(1, H), wn1p, ws1p, b1p)
    (acc1,) = _sc_agg(hw1, src_p.reshape(-1, 128), dst_p.reshape(-1, 128),
                      with_deg=False, mch=8, nbuf=4, g=128)
    out64 = _tc_final(hs1, acc1, deg)
    return out64[:, :C_OUT]


# speedup vs baseline: 1.4570x; 1.2707x over previous
import jax
import jax.numpy as jnp
from jax import lax
from jax.experimental import pallas as pl
from jax.experimental.pallas import tpu as pltpu
from jax.experimental.pallas import tpu_sc as plsc

N = 10000
D = 128
H = 128
C_OUT = 47
E = 320000
W2 = 64

NC = 2
NS = 16
NW = NC * NS
EPW = 10240
E_PAD = NW * EPW
N_PAD = 10240
RT = N_PAD // NS


def _fill(ref, nrows, ncols, value):
    v = jnp.full((16,), value, dtype=jnp.float32)
    nchunk = ncols // 16

    def body(i, carry):
        ref[i // nchunk, pl.ds((i % nchunk) * 16, 16)] = v
        return carry

    lax.fori_loop(0, nrows * nchunk, body, 0)


def _sc_agg(table, src2, dst2, with_deg, mch, nbuf, g):
    width = table.shape[1]
    rpw = EPW // g
    nm = rpw // mch
    nmp = nm // 2

    def body(*refs):
        if with_deg:
            (src2_h, dst2_h, table_h, acc_o, deg_o,
             src_v, dst_v, rows_v, acc_sh) = refs[:9]
            gsem = refs[9:9 + nbuf]
            ssem = refs[9 + nbuf:9 + 2 * nbuf]
            isem = refs[9 + 2 * nbuf]
            ones_v, deg_sh, dsem = refs[10 + 2 * nbuf:]
        else:
            (src2_h, dst2_h, table_h, acc_o,
             src_v, dst_v, rows_v, acc_sh) = refs[:8]
            gsem = refs[8:8 + nbuf]
            ssem = refs[8 + nbuf:8 + 2 * nbuf]
            isem = refs[8 + 2 * nbuf]
        cid = lax.axis_index("c")
        sid = lax.axis_index("s")
        wid = cid * NS + sid

        def gather(idx_row, p):
            pltpu.async_copy(table_h.at[idx_row], rows_v.at[p], gsem[p])

        def wait_gather(p):
            pltpu.make_async_copy(table_h.at[src_v.at[0, 0]],
                                  rows_v.at[p], gsem[p]).wait()

        def scatter(p, idx_row):
            pltpu.async_copy(rows_v.at[p], acc_sh.at[idx_row], ssem[p],
                             add=True)

        def wait_scatter(p):
            pltpu.make_async_copy(rows_v.at[p], acc_sh.at[dst_v.at[0, 0]],
                                  ssem[p]).wait()

        def drain_deg():
            for _ in range(mch):
                pltpu.make_async_copy(ones_v, deg_sh.at[dst_v.at[0, 0]],
                                      dsem).wait()

        _fill(rows_v.at[0], g, width, 0.0)
        for k in range(RT // g):
            pltpu.sync_copy(rows_v.at[0], acc_sh.at[pl.ds(sid * RT + k * g, g)])
        if with_deg:
            _fill(ones_v, g, 16, 0.0)
            for k in range(RT // g):
                pltpu.sync_copy(ones_v, deg_sh.at[pl.ds(sid * RT + k * g, g)])
            _fill(ones_v, g, 16, 1.0)

        plsc.subcore_barrier()

        def idx_off(m):
            return pl.multiple_of(wid * rpw + m * mch, 4)

        pltpu.sync_copy(src2_h.at[pl.ds(idx_off(0), mch)], src_v.at[0])
        pltpu.sync_copy(dst2_h.at[pl.ds(idx_off(0), mch)], dst_v.at[0])
        for j in range(nbuf - 1):
            gather(src_v.at[0, j], j)

        def half(t, hf):
            m = 2 * t + hf

            def drain_prev():
                wait_scatter(nbuf - 1)
                if with_deg:
                    drain_deg()

            if hf == 0:
                pl.when(t > 0)(drain_prev)
            else:
                drain_prev()

            def prefetch():
                pltpu.async_copy(src2_h.at[pl.ds(idx_off(m + 1), mch)],
                                 src_v.at[1 - hf], isem)
                pltpu.async_copy(dst2_h.at[pl.ds(idx_off(m + 1), mch)],
                                 dst_v.at[1 - hf], isem)

            if hf == 1:
                pl.when(t < nmp - 1)(prefetch)
            else:
                prefetch()

            for j in range(mch):
                p = j % nbuf
                wait_gather(p)
                scatter(p, dst_v.at[hf, j])
                if with_deg:
                    pltpu.async_copy(ones_v, deg_sh.at[dst_v.at[hf, j]],
                                     dsem, add=True)
                q = (p + nbuf - 1) % nbuf
                if j > 0:
                    wait_scatter(q)
                jn = j + nbuf - 1
                if jn < mch:
                    gather(src_v.at[hf, jn], q)
                else:
                    if jn == mch:
                        def idx_arrived():
                            pltpu.make_async_copy(
                                src2_h.at[pl.ds(idx_off(m + 1), mch)],
                                src_v.at[1 - hf], isem).wait()
                            pltpu.make_async_copy(
                                dst2_h.at[pl.ds(idx_off(m + 1), mch)],
                                dst_v.at[1 - hf], isem).wait()
                        if hf == 1:
                            pl.when(t < nmp - 1)(idx_arrived)
                        else:
                            idx_arrived()

                    def boundary():
                        gather(src_v.at[1 - hf, jn - mch], q)
                    if hf == 1:
                        pl.when(t < nmp - 1)(boundary)
                    else:
                        boundary()

        def pair(t, carry):
            half(t, 0)
            half(t, 1)
            return carry

        lax.fori_loop(0, nmp, pair, 0)

        wait_scatter(nbuf - 1)
        if with_deg:
            drain_deg()

        plsc.subcore_barrier()

        pltpu.sync_copy(acc_sh.at[pl.ds(sid * RT, RT)],
                        acc_o.at[cid, pl.ds(sid * RT, RT)])
        if with_deg:
            pltpu.sync_copy(deg_sh.at[pl.ds(sid * RT, RT)],
                            deg_o.at[cid, pl.ds(sid * RT, RT)])

    out_type = [jax.ShapeDtypeStruct((NC, N_PAD, width), jnp.float32)]
    scratch = ([pltpu.VMEM((2, mch, g), jnp.int32),
                pltpu.VMEM((2, mch, g), jnp.int32),
                pltpu.VMEM((nbuf, g, width), jnp.float32),
                pltpu.VMEM_SHARED((N_PAD, width), jnp.float32)]
               + [pltpu.SemaphoreType.DMA] * (2 * nbuf + 1))
    if with_deg:
        out_type.append(jax.ShapeDtypeStruct((NC, N_PAD, 16), jnp.float32))
        scratch += [pltpu.VMEM((g, 16), jnp.float32),
                    pltpu.VMEM_SHARED((N_PAD, 16), jnp.float32),
                    pltpu.SemaphoreType.DMA]

    fn = pl.kernel(
        body,
        out_type=out_type,
        mesh=plsc.VectorSubcoreMesh(core_axis_name="c", subcore_axis_name="s"),
        scratch_types=scratch,
        compiler_params=pltpu.CompilerParams(use_tc_tiling_on_sc=False),
    )
    return fn(src2, dst2, table)


R = 400
NB = N // R


def _tc_mid(x, acc, deg, ws0, wn0, b0, wn1p, ws1p, b1p):
    def body(x_r, a_r, d_r, ws0_r, wn0_r, b0_r, wn_r, ws_r, b_r, hw_o, hs_o):
        degv = d_r[0, :, 0:1] + d_r[1, :, 0:1]
        mean = (a_r[0] + a_r[1]) / jnp.maximum(degv, 1.0)
        h = (jnp.dot(x_r[...], ws0_r[...], preferred_element_type=jnp.float32)
             + jnp.dot(mean, wn0_r[...], preferred_element_type=jnp.float32)
             + b0_r[...])
        h = jnp.maximum(h, 0.0)
        nrm = jnp.sqrt(jnp.sum(h * h, axis=1, keepdims=True))
        h = h / jnp.maximum(nrm, 1e-12)
        hw_o[...] = jnp.dot(h, wn_r[...], preferred_element_type=jnp.float32)
        hs_o[...] = jnp.dot(h, ws_r[...],
                            preferred_element_type=jnp.float32) + b_r[...]

    return pl.pallas_call(
        body,
        grid=(NB,),
        in_specs=[pl.BlockSpec((R, D), lambda i: (i, 0)),
                  pl.BlockSpec((NC, R, D), lambda i: (0, i, 0)),
                  pl.BlockSpec((NC, R, 16), lambda i: (0, i, 0)),
                  pl.BlockSpec((D, H), lambda i: (0, 0)),
                  pl.BlockSpec((D, H), lambda i: (0, 0)),
                  pl.BlockSpec((1, H), lambda i: (0, 0)),
                  pl.BlockSpec((H, W2), lambda i: (0, 0)),
                  pl.BlockSpec((H, W2), lambda i: (0, 0)),
                  pl.BlockSpec((1, W2), lambda i: (0, 0))],
        out_specs=[pl.BlockSpec((R, W2), lambda i: (i, 0)),
                   pl.BlockSpec((R, W2), lambda i: (i, 0))],
        out_shape=[jax.ShapeDtypeStruct((N, W2), jnp.float32),
                   jax.ShapeDtypeStruct((N, W2), jnp.float32)],
    )(x, acc, deg, ws0, wn0, b0, wn1p, ws1p, b1p)


def _tc_final(hs1, acc, deg):
    def body(s_r, a_r, d_r, o_r):
        degv = d_r[0, :, 0:1] + d_r[1, :, 0:1]
        o_r[...] = s_r[...] + (a_r[0] + a_r[1]) / jnp.maximum(degv, 1.0)

    return pl.pallas_call(
        body,
        grid=(NB,),
        in_specs=[pl.BlockSpec((R, W2), lambda i: (i, 0)),
                  pl.BlockSpec((NC, R, W2), lambda i: (0, i, 0)),
                  pl.BlockSpec((NC, R, 16), lambda i: (0, i, 0))],
        out_specs=pl.BlockSpec((R, W2), lambda i: (i, 0)),
        out_shape=jax.ShapeDtypeStruct((N, W2), jnp.float32),
    )(hs1, acc, deg)


def kernel(features, edge_index, W_self0, W_neigh0, b0, W_self1, W_neigh1, b1):
    ppw = (E_PAD - E) // NW
    pad_src = jnp.zeros((NW, ppw), jnp.int32)
    pad_dst = jnp.broadcast_to(N + jnp.arange(ppw, dtype=jnp.int32),
                               (NW, ppw))
    src_p = jnp.concatenate([edge_index[0].reshape(NW, E // NW), pad_src],
                            axis=1)
    dst_p = jnp.concatenate([edge_index[1].reshape(NW, E // NW), pad_dst],
                            axis=1)
    wn1p = jnp.zeros((H, W2), jnp.float32).at[:, :C_OUT].set(W_neigh1)
    ws1p = jnp.zeros((H, W2), jnp.float32).at[:, :C_OUT].set(W_self1)
    b1p = jnp.zeros((1, W2), jnp.float32).at[0, :C_OUT].set(b1)

    acc0, deg = _sc_agg(features, src_p.reshape(-1, 64), dst_p.reshape(-1, 64),
                        with_deg=True, mch=16, nbuf=4, g=64)
    hw1, hs1 = _tc_mid(features, acc0, deg, W_self0, W_neigh0,
                       b0.reshape(1, H), wn1p, ws1p, b1p)
    (acc1,) = _sc_agg(hw1, src_p.reshape(-1, 128), dst_p.reshape(-1, 128),
                      with_deg=False, mch=16, nbuf=4, g=128)
    out64 = _tc_final(hs1, acc1, deg)
    return out64[:, :C_OUT]
